# baseline (speedup 1.0000x reference)
import functools

import jax
import jax.numpy as jnp
from jax import lax
from jax.experimental import pallas as pl
from jax.experimental.pallas import tpu as pltpu
from jax.experimental.pallas import tpu_sc as plsc

_B = 256
_L = 128
_HL = _L // 2
_D = 10000
_KB = 2560
_DP = 10240
_NK = _DP // _KB
_NA = 10
_PROJ = 256
_NPAIR = _NA * _NA * _HL

_NC = 2
_NS = 16
_NW = _NC * _NS
_MPW = _B // _NW
_MPC = _B // _NC
_NREG = _PROJ // 16


def _dot_nt(x, y):
    return lax.dot_general(x, y, (((1,), (1,)), ((), ())),
                           preferred_element_type=jnp.float32)


def _dot_nn(x, y):
    return lax.dot_general(x, y, (((1,), (0,)), ((), ())),
                           preferred_element_type=jnp.float32)


def _tc_body(idx_ref, rp_ref, a_ref, p_ref, tag_ref, w_ref, b_ref,
             h4_ref, part_ref, fidx_ref, h2_ref, kr_ref):
    k = pl.program_id(0)
    P = p_ref[...].astype(jnp.bfloat16)
    Wk = w_ref[...].astype(jnp.bfloat16)

    @pl.when(k == 0)
    def _():
        h2_ref[...] = jnp.zeros_like(h2_ref)
        part_ref[...] = jnp.broadcast_to(b_ref[0:1, :], (_B, _PROJ))
        idx = idx_ref[...]
        fidx_ref[...] = ((idx[:, :_HL] * _NA + idx[:, _HL:]) * _HL
                         + lax.broadcasted_iota(jnp.int32, (_B, _HL), 1))

    for a in range(_NA):
        kr_ref[a * _L:(a + 1) * _L, :] = P * a_ref[a:a + 1, :]
    h2_ref[...] += _dot_nt(kr_ref[...], Wk)

    iota_l = lax.broadcasted_iota(jnp.int32, (_B, _L), 1)
    ohi = (rp_ref[:, 0:1] == iota_l).astype(jnp.bfloat16)
    ohj = (rp_ref[:, 1:2] == iota_l).astype(jnp.bfloat16)
    pi = _dot_nn(ohi, P)
    pj = _dot_nn(ohj, P)
    r = (pi * (pj * tag_ref[...])).astype(jnp.bfloat16)
    part_ref[...] += _dot_nt(r, Wk)

    @pl.when(k == _NK - 1)
    def _():
        for a1 in range(_NA):
            blk1 = h2_ref[a1 * _L:a1 * _L + _HL, :]
            for a2 in range(_NA):
                blk2 = h2_ref[a2 * _L + _HL:(a2 + 1) * _L, :]
                r0 = (a1 * _NA + a2) * _HL
                h4_ref[r0:r0 + _HL, :] = blk1 + blk2


def _tc_stage(idx, rp, atom_p, pos, tag, W, b2):
    return pl.pallas_call(
        _tc_body,
        grid=(_NK,),
        in_specs=[
            pl.BlockSpec((_B, _L), lambda k: (0, 0)),
            pl.BlockSpec((_B, 2), lambda k: (0, 0)),
            pl.BlockSpec((16, _KB), lambda k: (0, k)),
            pl.BlockSpec((_L, _KB), lambda k: (0, k)),
            pl.BlockSpec((1, _KB), lambda k: (0, k)),
            pl.BlockSpec((_PROJ, _KB), lambda k: (0, k)),
            pl.BlockSpec((1, _PROJ), lambda k: (0, 0)),
        ],
        out_specs=[
            pl.BlockSpec((_NPAIR, _PROJ), lambda k: (0, 0)),
            pl.BlockSpec((_B, _PROJ), lambda k: (0, 0)),
            pl.BlockSpec((_B, _HL), lambda k: (0, 0)),
        ],
        out_shape=[
            jax.ShapeDtypeStruct((_NPAIR, _PROJ), jnp.float32),
            jax.ShapeDtypeStruct((_B, _PROJ), jnp.float32),
            jax.ShapeDtypeStruct((_B, _HL), jnp.int32),
        ],
        scratch_shapes=[
            pltpu.VMEM((_NA * _L, _PROJ), jnp.float32),
            pltpu.VMEM((_NA * _L, _KB), jnp.bfloat16),
        ],
    )(idx, rp, atom_p, pos, tag, W, b2)


def _sc_body(h4_hbm, fidx_hbm, part_hbm, out_hbm,
             fidx_v, rows_v, acc_v, sem0, sem1):
    c = lax.axis_index("c")
    s = lax.axis_index("s")
    gbase = c * _MPC + s * _MPW

    pltpu.sync_copy(part_hbm.at[pl.ds(gbase, _MPW)], acc_v)
    pltpu.sync_copy(fidx_hbm.at[pl.ds(gbase, _MPW)], fidx_v)

    sems = (sem0, sem1)
    cps = {0: pltpu.async_copy(h4_hbm.at[fidx_v.at[0]], rows_v.at[0], sem0)}
    for m in range(_MPW):
        if m + 1 < _MPW:
            nb = (m + 1) % 2
            cps[m + 1] = pltpu.async_copy(
                h4_hbm.at[fidx_v.at[m + 1]], rows_v.at[nb], sems[nb])
        cps.pop(m).wait()
        buf = rows_v.at[m % 2]

        def body(i, carry):
            l = i * 4
            for t in range(4):
                carry = tuple(cj + buf[l + t, pl.ds(j * 16, 16)]
                              for j, cj in enumerate(carry))
            return carry

        init = tuple(acc_v[m, pl.ds(j * 16, 16)] for j in range(_NREG))
        red = lax.fori_loop(0, _HL // 4, body, init)
        for j in range(_NREG):
            acc_v[m, pl.ds(j * 16, 16)] = red[j]

    pltpu.sync_copy(acc_v, out_hbm.at[pl.ds(gbase, _MPW)])


def _sc_stage(h4, fidx, part):
    mesh = plsc.VectorSubcoreMesh(core_axis_name="c", subcore_axis_name="s")
    f = functools.partial(
        pl.kernel,
        mesh=mesh,
        out_type=jax.ShapeDtypeStruct((_B, _PROJ), jnp.float32),
        scratch_types=[
            pltpu.VMEM((_MPW, _HL), jnp.int32),
            pltpu.VMEM((2, _HL, _PROJ), jnp.float32),
            pltpu.VMEM((_MPW, _PROJ), jnp.float32),
            pltpu.SemaphoreType.DMA,
            pltpu.SemaphoreType.DMA,
        ],
    )(_sc_body)
    return f(h4, fidx, part)


def kernel(atom_idx, ring_pairs, atom_hvs, pos_hvs, closure_tag, W, b):
    atom_p = jnp.pad(atom_hvs, ((0, 16 - _NA), (0, _DP - _D))
                     ).astype(jnp.bfloat16)
    tag2 = jnp.pad(closure_tag, (0, _DP - _D)).reshape(1, _DP)
    idx = atom_idx.astype(jnp.int32)
    rp = ring_pairs.astype(jnp.int32)
    b2 = b.reshape(1, _PROJ)

    h4, part, fidx = _tc_stage(idx, rp, atom_p, pos_hvs, tag2, W, b2)
    return _sc_stage(h4, fidx, part)


# speedup vs baseline: 1.0303x; 1.0303x over previous
import functools

import jax
import jax.numpy as jnp
from jax import lax
from jax.experimental import pallas as pl
from jax.experimental.pallas import tpu as pltpu
from jax.experimental.pallas import tpu_sc as plsc

_B = 256
_L = 128
_HL = _L // 2
_D = 10000
_KB = 2048
_DP = 10240
_NK = _DP // _KB
_NA = 10
_PROJ = 256
_NPAIR = _NA * _NA * _HL

_NC = 2
_NS = 16
_NW = _NC * _NS
_MPW = _B // _NW
_MPC = _B // _NC
_NREG = _PROJ // 16


def _dot_nt(x, y):
    return lax.dot_general(x, y, (((1,), (1,)), ((), ())),
                           preferred_element_type=jnp.float32)


def _dot_nn(x, y):
    return lax.dot_general(x, y, (((1,), (0,)), ((), ())),
                           preferred_element_type=jnp.float32)


def _tc_body(idx_ref, rp_ref, a_ref, p_ref, tag_ref, w_ref, b_ref,
             h4_ref, part_ref, fidx_ref, h2_ref, kr_ref):
    k = pl.program_id(0)
    P = p_ref[...]
    Wk = w_ref[...]

    @pl.when(k == 0)
    def _():
        h2_ref[...] = jnp.zeros_like(h2_ref)
        part_ref[...] = jnp.broadcast_to(b_ref[0:1, :], (_B, _PROJ))
        idx = idx_ref[...]
        fidx_ref[...] = ((idx[:, :_HL] * _NA + idx[:, _HL:]) * _HL
                         + lax.broadcasted_iota(jnp.int32, (_B, _HL), 1))

    for a in range(_NA):
        kr_ref[a * _L:(a + 1) * _L, :] = P * a_ref[a:a + 1, :]
    h2_ref[...] += _dot_nt(kr_ref[...], Wk)

    iota_l = lax.broadcasted_iota(jnp.int32, (_B, _L), 1)
    ohi = (rp_ref[:, 0:1] == iota_l).astype(jnp.bfloat16)
    ohj = (rp_ref[:, 1:2] == iota_l).astype(jnp.bfloat16)
    pi = _dot_nn(ohi, P)
    pj = _dot_nn(ohj, P)
    r = (pi * (pj * tag_ref[...])).astype(jnp.bfloat16)
    part_ref[...] += _dot_nt(r, Wk)

    @pl.when(k == _NK - 1)
    def _():
        for a1 in range(_NA):
            blk1 = h2_ref[a1 * _L:a1 * _L + _HL, :]
            for a2 in range(_NA):
                blk2 = h2_ref[a2 * _L + _HL:(a2 + 1) * _L, :]
                r0 = (a1 * _NA + a2) * _HL
                h4_ref[r0:r0 + _HL, :] = blk1 + blk2


def _tc_stage(idx, rp, atom_p, pos, tag, W, b2):
    return pl.pallas_call(
        _tc_body,
        grid=(_NK,),
        in_specs=[
            pl.BlockSpec((_B, _L), lambda k: (0, 0)),
            pl.BlockSpec((_B, 2), lambda k: (0, 0)),
            pl.BlockSpec((16, _KB), lambda k: (0, k)),
            pl.BlockSpec((_L, _KB), lambda k: (0, k)),
            pl.BlockSpec((1, _KB), lambda k: (0, k)),
            pl.BlockSpec((_PROJ, _KB), lambda k: (0, k)),
            pl.BlockSpec((1, _PROJ), lambda k: (0, 0)),
        ],
        out_specs=[
            pl.BlockSpec((_NPAIR, _PROJ), lambda k: (0, 0)),
            pl.BlockSpec((_B, _PROJ), lambda k: (0, 0)),
            pl.BlockSpec((_B, _HL), lambda k: (0, 0)),
        ],
        out_shape=[
            jax.ShapeDtypeStruct((_NPAIR, _PROJ), jnp.float32),
            jax.ShapeDtypeStruct((_B, _PROJ), jnp.float32),
            jax.ShapeDtypeStruct((_B, _HL), jnp.int32),
        ],
        scratch_shapes=[
            pltpu.VMEM((_NA * _L, _PROJ), jnp.float32),
            pltpu.VMEM((_NA * _L, _KB), jnp.bfloat16),
        ],
    )(idx, rp, atom_p, pos, tag, W, b2)


def _sc_body(h4_hbm, fidx_hbm, part_hbm, out_hbm,
             fidx_v, rows_v, acc_v, sem0, sem1):
    c = lax.axis_index("c")
    s = lax.axis_index("s")
    gbase = c * _MPC + s * _MPW

    pltpu.sync_copy(part_hbm.at[pl.ds(gbase, _MPW)], acc_v)
    pltpu.sync_copy(fidx_hbm.at[pl.ds(gbase, _MPW)], fidx_v)

    sems = (sem0, sem1)
    cps = {0: pltpu.async_copy(h4_hbm.at[fidx_v.at[0]], rows_v.at[0], sem0)}
    for m in range(_MPW):
        if m + 1 < _MPW:
            nb = (m + 1) % 2
            cps[m + 1] = pltpu.async_copy(
                h4_hbm.at[fidx_v.at[m + 1]], rows_v.at[nb], sems[nb])
        cps.pop(m).wait()
        buf = rows_v.at[m % 2]

        def body(i, carry):
            l = i * 4
            for t in range(4):
                carry = tuple(cj + buf[l + t, pl.ds(j * 16, 16)]
                              for j, cj in enumerate(carry))
            return carry

        init = tuple(acc_v[m, pl.ds(j * 16, 16)] for j in range(_NREG))
        red = lax.fori_loop(0, _HL // 4, body, init)
        for j in range(_NREG):
            acc_v[m, pl.ds(j * 16, 16)] = red[j]

    pltpu.sync_copy(acc_v, out_hbm.at[pl.ds(gbase, _MPW)])


def _sc_stage(h4, fidx, part):
    mesh = plsc.VectorSubcoreMesh(core_axis_name="c", subcore_axis_name="s")
    f = functools.partial(
        pl.kernel,
        mesh=mesh,
        out_type=jax.ShapeDtypeStruct((_B, _PROJ), jnp.float32),
        scratch_types=[
            pltpu.VMEM((_MPW, _HL), jnp.int32),
            pltpu.VMEM((2, _HL, _PROJ), jnp.float32),
            pltpu.VMEM((_MPW, _PROJ), jnp.float32),
            pltpu.SemaphoreType.DMA,
            pltpu.SemaphoreType.DMA,
        ],
    )(_sc_body)
    return f(h4, fidx, part)


def kernel(atom_idx, ring_pairs, atom_hvs, pos_hvs, closure_tag, W, b):
    atom_p = jnp.pad(atom_hvs, ((0, 16 - _NA), (0, _DP - _D))
                     ).astype(jnp.bfloat16)
    tag2 = jnp.pad(closure_tag, (0, _DP - _D)).reshape(1, _DP)
    pos_h = pos_hvs.astype(jnp.bfloat16)
    w_h = W.astype(jnp.bfloat16)
    idx = atom_idx.astype(jnp.int32)
    rp = ring_pairs.astype(jnp.int32)
    b2 = b.reshape(1, _PROJ)

    h4, part, fidx = _tc_stage(idx, rp, atom_p, pos_h, tag2, w_h, b2)
    return _sc_stage(h4, fidx, part)
